# baseline (speedup 1.0000x reference)
import functools

import jax
import jax.numpy as jnp
from jax.experimental import pallas as pl
from jax.experimental.pallas import tpu as pltpu

_TILE = 1024


def _body(a_ref, ann_ref, gcb_ref, gk_ref, gr_ref, gb_ref, dw_ref, db_ref,
          o_ref, *, nt):
    i = pl.program_id(0) % nt
    cc = ann_ref.shape[-1]
    a = a_ref[0].astype(jnp.bfloat16)
    ann = ann_ref[0].astype(jnp.bfloat16)
    x = jnp.dot(a, ann, preferred_element_type=jnp.float32) + gcb_ref[0]
    h = ann_ref[0, pl.ds(i * _TILE, _TILE), :]
    mx = jnp.dot(x, gk_ref[...], preferred_element_type=jnp.float32) + gb_ref[0]
    mi = jnp.dot(h, gr_ref[...], preferred_element_type=jnp.float32) + gb_ref[1]
    z = jax.nn.sigmoid(mx[:, :cc] + mi[:, :cc])
    r = jax.nn.sigmoid(mx[:, cc:2 * cc] + mi[:, cc:2 * cc])
    hh = jnp.tanh(mx[:, 2 * cc:] + r * mi[:, 2 * cc:])
    h_new = z * h + (1.0 - z) * hh
    o_ref[0] = jnp.dot(h_new, dw_ref[...],
                       preferred_element_type=jnp.float32) + db_ref[...]


def kernel(adjacent, annotations, gc_bias, gru_kernel, gru_recurrent,
           gru_bias, dense_w, dense_b):
    b, n, _ = adjacent.shape
    c = annotations.shape[-1]
    out_ch = dense_w.shape[-1]

    gc_bias2 = gc_bias.reshape(1, c)
    dense_b2 = dense_b.reshape(1, out_ch)

    nt = n // _TILE
    grid = (b * nt,)
    return pl.pallas_call(
        functools.partial(_body, nt=nt),
        grid=grid,
        in_specs=[
            pl.BlockSpec((1, _TILE, n), lambda j: (j // nt, j % nt, 0)),
            pl.BlockSpec((1, n, c), lambda j: (j // nt, 0, 0)),
            pl.BlockSpec((1, c), lambda j: (0, 0)),
            pl.BlockSpec(gru_kernel.shape, lambda j: (0, 0)),
            pl.BlockSpec(gru_recurrent.shape, lambda j: (0, 0)),
            pl.BlockSpec(gru_bias.shape, lambda j: (0, 0)),
            pl.BlockSpec(dense_w.shape, lambda j: (0, 0)),
            pl.BlockSpec((1, out_ch), lambda j: (0, 0)),
        ],
        out_specs=pl.BlockSpec((1, _TILE, out_ch),
                               lambda j: (j // nt, j % nt, 0)),
        out_shape=jax.ShapeDtypeStruct((b, n, out_ch), jnp.float32),
        compiler_params=pltpu.CompilerParams(
            dimension_semantics=("arbitrary",),
        ),
    )(adjacent, annotations, gc_bias2, gru_kernel, gru_recurrent,
      gru_bias, dense_w, dense_b2)


# speedup vs baseline: 1.0048x; 1.0048x over previous
import functools

import jax
import jax.numpy as jnp
from jax.experimental import pallas as pl
from jax.experimental.pallas import tpu as pltpu

_TILE = 1024


def _body(a_ref, ann_ref, gcb_ref, gk_ref, gr_ref, gb_ref, dw_ref, db_ref,
          o_ref, *, nt):
    i = pl.program_id(0) % nt
    cc = ann_ref.shape[-1]
    a = a_ref[0]
    ann = ann_ref[0]
    x = jnp.dot(a, ann, preferred_element_type=jnp.float32) + gcb_ref[0]
    h = ann_ref[0, pl.ds(i * _TILE, _TILE), :]
    mx = jnp.dot(x, gk_ref[...], preferred_element_type=jnp.float32) + gb_ref[0]
    mi = jnp.dot(h, gr_ref[...], preferred_element_type=jnp.float32) + gb_ref[1]
    z = jax.nn.sigmoid(mx[:, :cc] + mi[:, :cc])
    r = jax.nn.sigmoid(mx[:, cc:2 * cc] + mi[:, cc:2 * cc])
    hh = jnp.tanh(mx[:, 2 * cc:] + r * mi[:, 2 * cc:])
    h_new = z * h + (1.0 - z) * hh
    o_ref[0] = jnp.dot(h_new, dw_ref[...],
                       preferred_element_type=jnp.float32) + db_ref[...]


def kernel(adjacent, annotations, gc_bias, gru_kernel, gru_recurrent,
           gru_bias, dense_w, dense_b):
    b, n, _ = adjacent.shape
    c = annotations.shape[-1]
    out_ch = dense_w.shape[-1]

    gc_bias2 = gc_bias.reshape(1, c)
    dense_b2 = dense_b.reshape(1, out_ch)

    nt = n // _TILE
    grid = (b * nt,)
    return pl.pallas_call(
        functools.partial(_body, nt=nt),
        grid=grid,
        in_specs=[
            pl.BlockSpec((1, _TILE, n), lambda j: (j // nt, j % nt, 0)),
            pl.BlockSpec((1, n, c), lambda j: (j // nt, 0, 0)),
            pl.BlockSpec((1, c), lambda j: (0, 0)),
            pl.BlockSpec(gru_kernel.shape, lambda j: (0, 0)),
            pl.BlockSpec(gru_recurrent.shape, lambda j: (0, 0)),
            pl.BlockSpec(gru_bias.shape, lambda j: (0, 0)),
            pl.BlockSpec(dense_w.shape, lambda j: (0, 0)),
            pl.BlockSpec((1, out_ch), lambda j: (0, 0)),
        ],
        out_specs=pl.BlockSpec((1, _TILE, out_ch),
                               lambda j: (j // nt, j % nt, 0)),
        out_shape=jax.ShapeDtypeStruct((b, n, out_ch), jnp.float32),
        compiler_params=pltpu.CompilerParams(
            dimension_semantics=("arbitrary",),
        ),
    )(adjacent, annotations, gc_bias2, gru_kernel, gru_recurrent,
      gru_bias, dense_w, dense_b2)
